# Initial kernel scaffold; baseline (speedup 1.0000x reference)
import jax
import jax.numpy as jnp
from jax.experimental import pallas as pl


def kernel(x, W1, b1, W2, b2, Wg, bg):
    raise NotImplementedError("write your pallas kernel here")


# speedup vs baseline: 1.6610x; 1.6610x over previous
import functools

import jax
import jax.numpy as jnp
from jax import lax
from jax.experimental import pallas as pl
from jax.experimental.pallas import tpu as pltpu
from jax.experimental.pallas import tpu_sc as plsc

B, N, D = 2, 2048, 1024
E, TOPK, DFF = 8, 2, 2 * 1024
T = B * N
P = T * TOPK
EPAD = 128
TILE = 256
MAXT = P // TILE + E
PADT = MAXT * TILE

NC, NS = 2, 16
NW = NC * NS
TPW = T // NW
SUB = 32
NSUB = TPW // SUB


def _router_body(x_ref, wg_ref, bg_ref, p0_ref, p1_ref, g0_ref, g1_ref,
                 cnt_ref):
    x = x_ref[...]
    logits = jnp.dot(x, wg_ref[...], preferred_element_type=jnp.float32)
    logits = logits + bg_ref[...]
    lane = lax.broadcasted_iota(jnp.int32, (T, EPAD), 1)
    logits = jnp.where(lane < E, logits, -1e30)
    m0 = jnp.max(logits, axis=1, keepdims=True)
    a0 = jnp.min(jnp.where(logits == m0, lane, EPAD), axis=1, keepdims=True)
    oh0 = lane == a0
    l2 = jnp.where(oh0, -1e30, logits)
    m1 = jnp.max(l2, axis=1, keepdims=True)
    a1 = jnp.min(jnp.where(l2 == m1, lane, EPAD), axis=1, keepdims=True)
    oh1 = lane == a1
    e1 = jnp.exp(m1 - m0)
    den = 1.0 + e1
    g0_ref[...] = 1.0 / den
    g1_ref[...] = e1 / den
    oh = jnp.concatenate([oh0.astype(jnp.int32), oh1.astype(jnp.int32)],
                         axis=0)
    c = oh
    s = 1
    while s < P:
        c = c + jnp.concatenate(
            [jnp.zeros((s, EPAD), jnp.int32), c[:-s]], axis=0)
        s *= 2
    counts = c[P - 1:P, :]
    cnt_ref[...] = counts
    tiles = (counts + (TILE - 1)) // TILE
    padded = (tiles * TILE).astype(jnp.float32)
    r = lax.broadcasted_iota(jnp.int32, (EPAD, EPAD), 0)
    col = lax.broadcasted_iota(jnp.int32, (EPAD, EPAD), 1)
    tri = (r < col).astype(jnp.float32)
    poffs = jnp.dot(padded, tri,
                    preferred_element_type=jnp.float32).astype(jnp.int32)
    slot = (c - oh) + poffs
    p0_ref[...] = jnp.sum(jnp.where(oh0, slot[:T], 0), axis=1, keepdims=True)
    p1_ref[...] = jnp.sum(jnp.where(oh1, slot[T:], 0), axis=1, keepdims=True)


def _router(x_flat, wg_pad, bg_pad):
    return pl.pallas_call(
        _router_body,
        out_shape=(
            jax.ShapeDtypeStruct((T, 1), jnp.int32),
            jax.ShapeDtypeStruct((T, 1), jnp.int32),
            jax.ShapeDtypeStruct((T, 1), jnp.float32),
            jax.ShapeDtypeStruct((T, 1), jnp.float32),
            jax.ShapeDtypeStruct((1, EPAD), jnp.int32),
        ),
    )(x_flat, wg_pad, bg_pad)


def _scatter_body(x_hbm, p0_hbm, p1_hbm, xs_hbm, rows_v, i0_v, i1_v):
    wid = lax.axis_index("s") * NC + lax.axis_index("c")
    for s in range(NSUB):
        base = wid * TPW + s * SUB
        pltpu.sync_copy(x_hbm.at[pl.ds(base, SUB)], rows_v)
        pltpu.sync_copy(p0_hbm.at[pl.ds(base, SUB)], i0_v)
        pltpu.sync_copy(p1_hbm.at[pl.ds(base, SUB)], i1_v)
        pltpu.sync_copy(rows_v, xs_hbm.at[i0_v])
        pltpu.sync_copy(rows_v, xs_hbm.at[i1_v])


def _scatter(x_flat, perm0, perm1):
    mesh = plsc.VectorSubcoreMesh(core_axis_name="c", subcore_axis_name="s")
    f = functools.partial(
        pl.kernel,
        mesh=mesh,
        out_type=jax.ShapeDtypeStruct((PADT, D), jnp.float32),
        scratch_types=[
            pltpu.VMEM((SUB, D), jnp.float32),
            pltpu.VMEM((SUB,), jnp.int32),
            pltpu.VMEM((SUB,), jnp.int32),
        ],
    )(_scatter_body)
    return f(x_flat, perm0, perm1)


def _ffn_body(texp_ref, x_ref, w1_ref, b1_ref, w2_ref, b2_ref, y_ref):
    del texp_ref
    x = x_ref[...]
    h = jnp.dot(x, w1_ref[0], preferred_element_type=jnp.float32)
    h = jnp.maximum(h + b1_ref[0], 0.0)
    y = jnp.dot(h, w2_ref[0], preferred_element_type=jnp.float32)
    y_ref[...] = y + b2_ref[0]


def _ffn(texp, x_sorted, W1, b1, W2, b2):
    grid_spec = pltpu.PrefetchScalarGridSpec(
        num_scalar_prefetch=1,
        grid=(MAXT,),
        in_specs=[
            pl.BlockSpec((TILE, D), lambda i, s: (i, 0)),
            pl.BlockSpec((1, D, DFF), lambda i, s: (s[i], 0, 0)),
            pl.BlockSpec((1, 1, DFF), lambda i, s: (s[i], 0, 0)),
            pl.BlockSpec((1, DFF, D), lambda i, s: (s[i], 0, 0)),
            pl.BlockSpec((1, 1, D), lambda i, s: (s[i], 0, 0)),
        ],
        out_specs=pl.BlockSpec((TILE, D), lambda i, s: (i, 0)),
    )
    return pl.pallas_call(
        _ffn_body,
        grid_spec=grid_spec,
        out_shape=jax.ShapeDtypeStruct((PADT, D), jnp.float32),
        compiler_params=pltpu.CompilerParams(
            dimension_semantics=("arbitrary",)),
    )(texp, x_sorted, W1, b1.reshape(E, 1, DFF), W2, b2.reshape(E, 1, D))


def _combine_body(y_hbm, p0_hbm, p1_hbm, g0_hbm, g1_hbm, out_hbm,
                  r0_v, r1_v, i0_v, i1_v, g0_v, g1_v, sem0, sem1):
    wid = lax.axis_index("s") * NC + lax.axis_index("c")
    for s in range(NSUB):
        base = wid * TPW + s * SUB
        pltpu.sync_copy(p0_hbm.at[pl.ds(base, SUB)], i0_v)
        pltpu.sync_copy(p1_hbm.at[pl.ds(base, SUB)], i1_v)
        pltpu.sync_copy(g0_hbm.at[pl.ds(base, SUB)], g0_v)
        pltpu.sync_copy(g1_hbm.at[pl.ds(base, SUB)], g1_v)
        cp0 = pltpu.async_copy(y_hbm.at[i0_v], r0_v, sem0)
        cp1 = pltpu.async_copy(y_hbm.at[i1_v], r1_v, sem1)
        cp0.wait()
        cp1.wait()

        for h in range(SUB // 16):
            gv0 = g0_v[pl.ds(h * 16, 16)]
            gv1 = g1_v[pl.ds(h * 16, 16)]

            def body(j, carry, gv0=gv0, gv1=gv1, h=h):
                idx = (jnp.full((16,), 0, jnp.int32) + j)[:, None]
                dn = lax.GatherDimensionNumbers(
                    offset_dims=(), collapsed_slice_dims=(0,),
                    start_index_map=(0,))
                s0 = lax.gather(
                    gv0, idx, dn, (1,),
                    mode=lax.GatherScatterMode.PROMISE_IN_BOUNDS)
                s1 = lax.gather(
                    gv1, idx, dn, (1,),
                    mode=lax.GatherScatterMode.PROMISE_IN_BOUNDS)
                row = h * 16 + j
                for cb in range(D // 16):
                    a = r0_v[row, pl.ds(cb * 16, 16)]
                    b = r1_v[row, pl.ds(cb * 16, 16)]
                    r0_v[row, pl.ds(cb * 16, 16)] = s0 * a + s1 * b
                return carry

            lax.fori_loop(0, 16, body, 0)
        pltpu.sync_copy(r0_v, out_hbm.at[pl.ds(base, SUB)])


def _combine(y_sorted, perm0, perm1, g0, g1):
    mesh = plsc.VectorSubcoreMesh(core_axis_name="c", subcore_axis_name="s")
    f = functools.partial(
        pl.kernel,
        mesh=mesh,
        out_type=jax.ShapeDtypeStruct((T, D), jnp.float32),
        scratch_types=[
            pltpu.VMEM((SUB, D), jnp.float32),
            pltpu.VMEM((SUB, D), jnp.float32),
            pltpu.VMEM((SUB,), jnp.int32),
            pltpu.VMEM((SUB,), jnp.int32),
            pltpu.VMEM((SUB,), jnp.float32),
            pltpu.VMEM((SUB,), jnp.float32),
            pltpu.SemaphoreType.DMA,
            pltpu.SemaphoreType.DMA,
        ],
    )(_combine_body)
    return f(y_sorted, perm0, perm1, g0, g1)


def kernel(x, W1, b1, W2, b2, Wg, bg):
    x_flat = x.reshape(T, D)
    wg_pad = jnp.zeros((D, EPAD), jnp.float32).at[:, :E].set(Wg)
    bg_pad = jnp.zeros((1, EPAD), jnp.float32).at[0, :E].set(bg)
    p0, p1, g0, g1, cnt = _router(x_flat, wg_pad, bg_pad)
    perm0 = p0.reshape(T)
    perm1 = p1.reshape(T)
    g0 = g0.reshape(T)
    g1 = g1.reshape(T)
    counts = cnt.reshape(EPAD)[:E]
    tiles_e = (counts + (TILE - 1)) // TILE
    bounds = jnp.cumsum(tiles_e)
    texp = jnp.sum((jnp.arange(MAXT)[:, None] >= bounds[None, :])
                   .astype(jnp.int32), axis=1)
    texp = jnp.minimum(texp, E - 1).astype(jnp.int32)

    x_sorted = _scatter(x_flat, perm0, perm1)
    y_sorted = _ffn(texp, x_sorted, W1, b1, W2, b2)
    out = _combine(y_sorted, perm0, perm1, g0, g1)
    return out.reshape(B, N, D)
